# baseline (speedup 1.0000x reference)
import numpy as np
import jax
import jax.numpy as jnp
from jax.experimental import pallas as pl

BS, K, N = 16, 8, 128
H, W = 256, 256
OUT_H, OUT_W = 512, 256
C = BS * K

CPB = 16

_NEG = np.float32(-3e38)
_POS = np.float32(3e38)


def _vq_row(tp_ref, vp_ref, mm_ref, cur, Q, idx, q):
    t0 = tp_ref[cur]
    v0 = vp_ref[cur]
    vmin = mm_ref[cur, 0, 0]
    vmax = mm_ref[cur, 0, 1]
    step_t = np.float32(1.0 / (H - 1))
    t = t0 / step_t
    step_v = (vmax - vmin) / np.float32(W - 1)
    v = (v0 - vmin) / step_v
    ts = jnp.where(t > 0.0, t, np.float32(1e9))

    T = jnp.broadcast_to(ts, (N, H))
    V = jnp.broadcast_to(v, (N, H))

    le = T <= Q
    t_lo = jnp.max(jnp.where(le, T, _NEG), axis=0, keepdims=True)
    i_lo = jnp.max(jnp.where(le & (T == t_lo), idx, -1), axis=0, keepdims=True)
    v_lo = jnp.sum(jnp.where(idx == i_lo, V, 0.0), axis=0, keepdims=True)
    gt = jnp.logical_not(le)
    t_hi = jnp.min(jnp.where(gt, T, _POS), axis=0, keepdims=True)
    i_hi = jnp.min(jnp.where(gt & (T == t_hi), idx, N + 1), axis=0, keepdims=True)
    v_hi = jnp.sum(jnp.where(idx == i_hi, V, 0.0), axis=0, keepdims=True)

    interp = v_lo + (q - t_lo) / (t_hi - t_lo) * (v_hi - v_lo)
    vq = jnp.where(t_lo == _NEG, v_hi, jnp.where(t_hi == _POS, v_lo, interp))
    return jnp.clip(jnp.round(vq), 0, W - 1).astype(jnp.int32)


def _cv_kernel(tp_ref, vp_ref, mm_ref, out_ref):
    step = pl.program_id(0)
    Q = jax.lax.broadcasted_iota(jnp.int32, (N, H), 1).astype(jnp.float32)
    idx = jax.lax.broadcasted_iota(jnp.int32, (N, H), 0)
    q = jax.lax.broadcasted_iota(jnp.int32, (1, H), 1).astype(jnp.float32)
    cols = jax.lax.broadcasted_iota(jnp.int32, (H, W), 1)

    base = np.float32(0.01)
    hi_w = np.float32(0.675 + 0.01)
    lo_w = np.float32(0.225)
    vrows = [
        _vq_row(tp_ref, vp_ref, mm_ref, step * CPB + c, Q, idx, q)
        for c in range(CPB)
    ]
    viTs = [
        jnp.swapaxes(jnp.broadcast_to(vrow, (8, H)), 0, 1) for vrow in vrows
    ]
    for c in range(CPB):
        vi = viTs[c][:, 0:1]
        vi_prev = jnp.concatenate([vi[0:1], vi[:-1]], axis=0)
        vi_next = jnp.concatenate([vi[1:], vi[-1:]], axis=0)
        hitb = jnp.where(cols == vi, hi_w, base)
        even = hitb + jnp.where(cols == vi_prev, lo_w, np.float32(0.0))
        odd = hitb + jnp.where(cols == vi_next, lo_w, np.float32(0.0))
        out_ref[c] = jnp.concatenate([even, odd], axis=1)


@jax.jit
def kernel(VelPoints, VMM):
    P = VelPoints.reshape(C, N, 2)
    tp = P[:, :, 0:1]
    vp = P[:, :, 1:2]
    mm = jnp.repeat(VMM, K, axis=0)[:, None, :]
    out = pl.pallas_call(
        _cv_kernel,
        grid=(C // CPB,),
        in_specs=[
            pl.BlockSpec((C, N, 1), lambda i: (0, 0, 0)),
            pl.BlockSpec((C, N, 1), lambda i: (0, 0, 0)),
            pl.BlockSpec((C, 1, 2), lambda i: (0, 0, 0)),
        ],
        out_specs=pl.BlockSpec((CPB, H, 2 * W), lambda i: (i, 0, 0)),
        out_shape=jax.ShapeDtypeStruct((C, H, 2 * W), jnp.float32),
    )(tp, vp, mm)
    return out.reshape(BS, K, OUT_H, OUT_W)


# speedup vs baseline: 1.1402x; 1.1402x over previous
import numpy as np
import jax
import jax.numpy as jnp
from jax.experimental import pallas as pl

BS, K, N = 16, 8, 128
H, W = 256, 256
OUT_H, OUT_W = 512, 256
C = BS * K

CPB = 8

_NEG = np.float32(-3e38)
_POS = np.float32(3e38)


def _vq_row(p_ref, mm_ref, cur, Q, idx, q):
    pts = p_ref[cur]
    t0 = pts[:, 0:1]
    v0 = pts[:, 1:2]
    b = cur // K
    vmin = mm_ref[b, 0, 0]
    vmax = mm_ref[b, 0, 1]
    step_t = np.float32(1.0 / (H - 1))
    t = t0 / step_t
    step_v = (vmax - vmin) / np.float32(W - 1)
    v = (v0 - vmin) / step_v
    ts = jnp.where(t > 0.0, t, np.float32(1e9))

    T = jnp.broadcast_to(ts, (N, H))
    V = jnp.broadcast_to(v, (N, H))

    le = T <= Q
    t_lo = jnp.max(jnp.where(le, T, _NEG), axis=0, keepdims=True)
    i_lo = jnp.max(jnp.where(le & (T == t_lo), idx, -1), axis=0, keepdims=True)
    v_lo = jnp.sum(jnp.where(idx == i_lo, V, 0.0), axis=0, keepdims=True)
    gt = jnp.logical_not(le)
    t_hi = jnp.min(jnp.where(gt, T, _POS), axis=0, keepdims=True)
    i_hi = jnp.min(jnp.where(gt & (T == t_hi), idx, N + 1), axis=0, keepdims=True)
    v_hi = jnp.sum(jnp.where(idx == i_hi, V, 0.0), axis=0, keepdims=True)

    interp = v_lo + (q - t_lo) / (t_hi - t_lo) * (v_hi - v_lo)
    vq = jnp.where(t_lo == _NEG, v_hi, jnp.where(t_hi == _POS, v_lo, interp))
    return jnp.clip(jnp.round(vq), 0, W - 1).astype(jnp.int32)


def _cv_kernel(p_ref, mm_ref, out_ref):
    step = pl.program_id(0)
    Q = jax.lax.broadcasted_iota(jnp.int32, (N, H), 1).astype(jnp.float32)
    idx = jax.lax.broadcasted_iota(jnp.int32, (N, H), 0)
    q = jax.lax.broadcasted_iota(jnp.int32, (1, H), 1).astype(jnp.float32)
    cols = jax.lax.broadcasted_iota(jnp.int32, (H, W), 1)

    base = np.float32(0.01)
    hi_w = np.float32(0.675 + 0.01)
    lo_w = np.float32(0.225)
    vrows = [
        _vq_row(p_ref, mm_ref, step * CPB + c, Q, idx, q) for c in range(CPB)
    ]
    viTs = [
        jnp.swapaxes(jnp.broadcast_to(vrow, (8, H)), 0, 1) for vrow in vrows
    ]
    for c in range(CPB):
        vi = viTs[c][:, 0:1]
        vi_prev = jnp.concatenate([vi[0:1], vi[:-1]], axis=0)
        vi_next = jnp.concatenate([vi[1:], vi[-1:]], axis=0)
        hitb = jnp.where(cols == vi, hi_w, base)
        even = hitb + jnp.where(cols == vi_prev, lo_w, np.float32(0.0))
        odd = hitb + jnp.where(cols == vi_next, lo_w, np.float32(0.0))
        out_ref[c] = jnp.concatenate([even, odd], axis=1)


@jax.jit
def kernel(VelPoints, VMM):
    P = VelPoints.reshape(C, N, 2)
    mm = VMM[:, None, :]
    out = pl.pallas_call(
        _cv_kernel,
        grid=(C // CPB,),
        in_specs=[
            pl.BlockSpec((C, N, 2), lambda i: (0, 0, 0)),
            pl.BlockSpec((BS, 1, 2), lambda i: (0, 0, 0)),
        ],
        out_specs=pl.BlockSpec((CPB, H, 2 * W), lambda i: (i, 0, 0)),
        out_shape=jax.ShapeDtypeStruct((C, H, 2 * W), jnp.float32),
    )(P, mm)
    return out.reshape(BS, K, OUT_H, OUT_W)


# speedup vs baseline: 1.1672x; 1.0236x over previous
import numpy as np
import jax
import jax.numpy as jnp
from jax.experimental import pallas as pl

BS, K, N = 16, 8, 128
H, W = 256, 256
OUT_H, OUT_W = 512, 256
C = BS * K

CPB = 8

_NEG = np.float32(-3e38)
_POS = np.float32(3e38)


def _vq_row(p_ref, mm_ref, cur, Q, idx, q):
    pts = p_ref[cur]
    t0 = pts[:, 0:1]
    v0 = pts[:, 1:2]
    b = cur // K
    vmin = mm_ref[b, 0, 0]
    vmax = mm_ref[b, 0, 1]
    step_t = np.float32(1.0 / (H - 1))
    t = t0 / step_t
    step_v = (vmax - vmin) / np.float32(W - 1)
    v = (v0 - vmin) / step_v
    ts = jnp.where(t > 0.0, t, np.float32(1e9))

    T = jnp.broadcast_to(ts, (N, H))
    V = jnp.broadcast_to(v, (N, H))

    le = T <= Q
    t_lo = jnp.max(jnp.where(le, T, _NEG), axis=0, keepdims=True)
    i_lo = jnp.max(jnp.where(T == t_lo, idx, -1), axis=0, keepdims=True)
    v_lo = jnp.sum(jnp.where(idx == i_lo, V, 0.0), axis=0, keepdims=True)
    t_hi = jnp.min(jnp.where(le, _POS, T), axis=0, keepdims=True)
    i_hi = jnp.min(jnp.where(T == t_hi, idx, N + 1), axis=0, keepdims=True)
    v_hi = jnp.sum(jnp.where(idx == i_hi, V, 0.0), axis=0, keepdims=True)

    interp = v_lo + (q - t_lo) / (t_hi - t_lo) * (v_hi - v_lo)
    vq = jnp.where(t_lo == _NEG, v_hi, jnp.where(t_hi == _POS, v_lo, interp))
    return jnp.clip(jnp.round(vq), 0, W - 1).astype(jnp.int32)


def _cv_kernel(p_ref, mm_ref, out_ref):
    step = pl.program_id(0)
    Q = jax.lax.broadcasted_iota(jnp.int32, (N, H), 1).astype(jnp.float32)
    idx = jax.lax.broadcasted_iota(jnp.int32, (N, H), 0)
    q = jax.lax.broadcasted_iota(jnp.int32, (1, H), 1).astype(jnp.float32)
    cols = jax.lax.broadcasted_iota(jnp.int32, (H, W), 1)

    base = np.float32(0.01)
    hi_w = np.float32(0.675 + 0.01)
    lo_w = np.float32(0.225)
    vrows = [
        _vq_row(p_ref, mm_ref, step * CPB + c, Q, idx, q) for c in range(CPB)
    ]
    viTs = [
        jnp.swapaxes(jnp.broadcast_to(vrow, (8, H)), 0, 1) for vrow in vrows
    ]
    for c in range(CPB):
        vi = viTs[c][:, 0:1]
        vi_prev = jnp.concatenate([vi[0:1], vi[:-1]], axis=0)
        vi_next = jnp.concatenate([vi[1:], vi[-1:]], axis=0)
        hitb = jnp.where(cols == vi, hi_w, base)
        even = hitb + jnp.where(cols == vi_prev, lo_w, np.float32(0.0))
        odd = hitb + jnp.where(cols == vi_next, lo_w, np.float32(0.0))
        out_ref[c] = jnp.concatenate([even, odd], axis=1)


@jax.jit
def kernel(VelPoints, VMM):
    P = VelPoints.reshape(C, N, 2)
    mm = VMM[:, None, :]
    out = pl.pallas_call(
        _cv_kernel,
        grid=(C // CPB,),
        in_specs=[
            pl.BlockSpec((C, N, 2), lambda i: (0, 0, 0)),
            pl.BlockSpec((BS, 1, 2), lambda i: (0, 0, 0)),
        ],
        out_specs=pl.BlockSpec((CPB, H, 2 * W), lambda i: (i, 0, 0)),
        out_shape=jax.ShapeDtypeStruct((C, H, 2 * W), jnp.float32),
    )(P, mm)
    return out.reshape(BS, K, OUT_H, OUT_W)
